# baseline (speedup 1.0000x reference)
import functools

import jax
import jax.numpy as jnp
from jax import lax
from jax.experimental import pallas as pl
from jax.experimental.pallas import tpu as pltpu
from jax.experimental.pallas import tpu_sc as plsc

N = 6400000
RANK = N // 10
NC, NS, L = 2, 16, 16
NW = NC * NS
PER_W = N // NW
CHUNK = 20000
NCHUNKS = PER_W // CHUNK
NB1 = 16384
NB2 = 16384

_mesh = plsc.VectorSubcoreMesh(core_axis_name="c", subcore_axis_name="s")


def _worker_id():
    return lax.axis_index("s") * NC + lax.axis_index("c")


def _zero_ref(ref, n):
    @plsc.parallel_loop(0, n, step=L, unroll=8)
    def _(i):
        ref[pl.ds(i, L)] = jnp.zeros((L,), jnp.int32)


def _stream_chunks(x_hbm, base, buf, sem0, sem1, process):
    sems = (sem0, sem1)
    cps = [None, None]
    cps[0] = pltpu.async_copy(x_hbm.at[0, pl.ds(base, CHUNK)], buf.at[0], sems[0])
    for c in range(NCHUNKS):
        cur = c & 1
        if c + 1 < NCHUNKS:
            cps[1 - cur] = pltpu.async_copy(
                x_hbm.at[0, pl.ds(base + (c + 1) * CHUNK, CHUNK)],
                buf.at[1 - cur],
                sems[1 - cur],
            )
        cps[cur].wait()
        process(c, cur)


@functools.partial(
    pl.kernel,
    out_type=jax.ShapeDtypeStruct((NW, NB1), jnp.int32),
    mesh=_mesh,
    compiler_params=pltpu.CompilerParams(needs_layout_passes=False, use_tc_tiling_on_sc=False),
    scratch_types=[
        pltpu.VMEM((2, CHUNK), jnp.float32),
        pltpu.VMEM((NB1,), jnp.int32),
        pltpu.SemaphoreType.DMA,
        pltpu.SemaphoreType.DMA,
    ],
)
def _sc_coarse_hist(x_hbm, out_hbm, buf, hist, sem0, sem1):
    wid = _worker_id()
    base = wid * PER_W
    _zero_ref(hist, NB1)

    def process(c, cur):
        @plsc.parallel_loop(0, CHUNK, step=L, unroll=8)
        def _(i):
            v = buf[cur, pl.ds(i, L)]
            bits = lax.bitcast_convert_type(v, jnp.int32)
            hi = lax.shift_right_logical(bits, 16)
            plsc.addupdate_scatter(hist, [hi], jnp.ones((L,), jnp.int32))

    _stream_chunks(x_hbm, base, buf, sem0, sem1, process)
    pltpu.sync_copy(hist, out_hbm.at[wid])


@functools.partial(
    pl.kernel,
    out_type=jax.ShapeDtypeStruct((NW, NB2), jnp.int32),
    mesh=_mesh,
    compiler_params=pltpu.CompilerParams(needs_layout_passes=False, use_tc_tiling_on_sc=False),
    scratch_types=[
        pltpu.VMEM((2, CHUNK), jnp.float32),
        pltpu.VMEM((NB2,), jnp.int32),
        pltpu.VMEM((L,), jnp.int32),
        pltpu.SemaphoreType.DMA,
        pltpu.SemaphoreType.DMA,
    ],
)
def _sc_fine_hist(x_hbm, cb_hbm, out_hbm, buf, hist, cbv, sem0, sem1):
    wid = _worker_id()
    base = wid * PER_W
    _zero_ref(hist, NB2)
    pltpu.sync_copy(cb_hbm, cbv)
    cb = cbv[...]
    ones = jnp.ones((L,), jnp.int32)
    lowmask = jnp.full((L,), NB2 - 1, jnp.int32)

    def process(c, cur):
        @plsc.parallel_loop(0, CHUNK, step=L, unroll=8)
        def _(i):
            v = buf[cur, pl.ds(i, L)]
            bits = lax.bitcast_convert_type(v, jnp.int32)
            hi = lax.shift_right_logical(bits, 16)
            lo = jnp.bitwise_and(lax.shift_right_logical(bits, 2), lowmask)
            plsc.addupdate_scatter(hist, [lo], ones, mask=hi == cb)

    _stream_chunks(x_hbm, base, buf, sem0, sem1, process)
    pltpu.sync_copy(hist, out_hbm.at[wid])


@functools.partial(
    pl.kernel,
    out_type=jax.ShapeDtypeStruct((N,), jnp.float32),
    mesh=_mesh,
    compiler_params=pltpu.CompilerParams(needs_layout_passes=False, use_tc_tiling_on_sc=False),
    scratch_types=[
        pltpu.VMEM((2, CHUNK), jnp.float32),
        pltpu.VMEM((2, CHUNK), jnp.float32),
        pltpu.VMEM((L,), jnp.int32),
        pltpu.SemaphoreType.DMA,
        pltpu.SemaphoreType.DMA,
        pltpu.SemaphoreType.DMA,
        pltpu.SemaphoreType.DMA,
    ],
)
def _sc_write_mask(x_hbm, tb_hbm, y_hbm, buf, obuf, tbv, sem0, sem1, osem0, osem1):
    wid = _worker_id()
    base = wid * PER_W
    pltpu.sync_copy(tb_hbm, tbv)
    tb = tbv[...]
    osems = (osem0, osem1)
    ocps = [None, None]

    def process(c, cur):
        if ocps[cur] is not None:
            ocps[cur].wait()

        @plsc.parallel_loop(0, CHUNK, step=L, unroll=8)
        def _(i):
            v = buf[cur, pl.ds(i, L)]
            bits = lax.bitcast_convert_type(v, jnp.int32)
            obuf[cur, pl.ds(i, L)] = (bits >= tb).astype(jnp.float32)
        ocps[cur] = pltpu.async_copy(
            obuf.at[cur], y_hbm.at[pl.ds(base + c * CHUNK, CHUNK)], osems[cur]
        )

    _stream_chunks(x_hbm, base, buf, sem0, sem1, process)
    ocps[0].wait()
    ocps[1].wait()


def _bisect(h, iota, rank, lo0, hi0, steps):
    def body(t, carry):
        lo, hi = carry
        mid = (lo + hi) // 2
        c = jnp.sum(jnp.where(iota > mid, h, 0))
        pred = c < rank
        return jnp.where(pred, lo, mid), jnp.where(pred, mid, hi)

    lo, hi = lax.fori_loop(0, steps, body, (jnp.int32(lo0), jnp.int32(hi0)))
    cnt = jnp.sum(jnp.where(iota > hi, h, 0))
    return hi, cnt


def _tc_coarse_body(h1_ref, out_ref):
    h = jnp.sum(h1_ref[...], axis=0, keepdims=True)
    iota = lax.broadcasted_iota(jnp.int32, (1, NB1), 1)
    cb, _ = _bisect(h, iota, RANK, -1, NB1 - 1, 14)
    out_ref[...] = jnp.full((L,), cb, jnp.int32)


def _tc_fine_body(h1_ref, h2_ref, cb_ref, out_ref):
    cb = jnp.max(cb_ref[...])
    h1 = jnp.sum(h1_ref[...], axis=0, keepdims=True)
    iota1 = lax.broadcasted_iota(jnp.int32, (1, NB1), 1)
    cnt_gt = jnp.sum(jnp.where(iota1 > cb, h1, 0))
    r1 = RANK - cnt_gt
    h2 = jnp.sum(h2_ref[...], axis=0, keepdims=True)
    iota2 = lax.broadcasted_iota(jnp.int32, (1, NB2), 1)
    fb, _ = _bisect(h2, iota2, r1, -1, NB2 - 1, 14)
    tb = lax.shift_left(cb, 16) + lax.shift_left(fb, 2)
    out_ref[...] = jnp.full((L,), tb, jnp.int32)


EX_GRID = 25
EX_BLK = N // EX_GRID


def _tc_extract_body(kref, b_ref, x_ref):
    x_ref[...] = b_ref[pl.ds(kref[0], 1), :]


def _tc_kld_body(x_ref, kld_ref):
    pi = pl.program_id(0)
    xc = jnp.clip(x_ref[...], -10.0, 10.0)
    u = jnp.exp(-xc)
    w = 1.0 + u
    t = 0.6931471805599453 - jnp.log(w) - xc * u / w
    s = jnp.sum(t) * (1.0 / N)
    prev = jnp.where(pi == 0, 0.0, kld_ref[0, 0])
    kld_ref[0, 0] = prev + s


def kernel(B, k):
    k_arr = jnp.reshape(k, (1,)).astype(jnp.int32)
    x = pl.pallas_call(
        _tc_extract_body,
        grid_spec=pltpu.PrefetchScalarGridSpec(
            num_scalar_prefetch=1,
            grid=(EX_GRID,),
            in_specs=[pl.BlockSpec((4, EX_BLK), lambda i, kref: (0, i))],
            out_specs=[pl.BlockSpec((1, EX_BLK), lambda i, kref: (0, i))],
        ),
        out_shape=[jax.ShapeDtypeStruct((1, N), jnp.float32)],
    )(k_arr, B)[0]
    KLD_SPLIT = 4
    QN = N // KLD_SPLIT
    QG = 4
    kparts = []
    for q in range(KLD_SPLIT):
        kparts.append(
            pl.pallas_call(
                _tc_kld_body,
                grid=(QG,),
                in_specs=[
                    pl.BlockSpec(
                        (1, QN // QG), lambda i, q=q: (0, q * QG + i)
                    )
                ],
                out_specs=pl.BlockSpec(
                    block_shape=(1, 1),
                    index_map=lambda i: (0, 0),
                    memory_space=pltpu.SMEM,
                ),
                out_shape=jax.ShapeDtypeStruct((1, 1), jnp.float32),
            )(x)
        )
    kld = kparts[0] + kparts[1] + kparts[2] + kparts[3]
    h1 = _sc_coarse_hist(x)
    cb = pl.pallas_call(
        _tc_coarse_body,
        out_shape=jax.ShapeDtypeStruct((L,), jnp.int32),
    )(h1)
    h2 = _sc_fine_hist(x, cb)
    tb = pl.pallas_call(
        _tc_fine_body,
        out_shape=jax.ShapeDtypeStruct((L,), jnp.int32),
    )(h1, h2, cb)
    y = _sc_write_mask(x, tb)
    return y, kld[0, 0]


# speedup vs baseline: 1.1446x; 1.1446x over previous
import functools

import jax
import jax.numpy as jnp
from jax import lax
from jax.experimental import pallas as pl
from jax.experimental.pallas import tpu as pltpu
from jax.experimental.pallas import tpu_sc as plsc

N = 6400000
RANK = N // 10
NC, NS, L = 2, 16, 16
NW = NC * NS
PER_W = N // NW
CHUNK = 20000
NCHUNKS = PER_W // CHUNK
NB1 = 16384
NB2 = 16384

_mesh = plsc.VectorSubcoreMesh(core_axis_name="c", subcore_axis_name="s")


def _worker_id():
    return lax.axis_index("s") * NC + lax.axis_index("c")


def _zero_ref(ref, n):
    @plsc.parallel_loop(0, n, step=L, unroll=8)
    def _(i):
        ref[pl.ds(i, L)] = jnp.zeros((L,), jnp.int32)


def _stream_chunks(x_hbm, base, buf, sem0, sem1, process):
    sems = (sem0, sem1)
    cps = [None, None]
    cps[0] = pltpu.async_copy(x_hbm.at[0, pl.ds(base, CHUNK)], buf.at[0], sems[0])
    for c in range(NCHUNKS):
        cur = c & 1
        if c + 1 < NCHUNKS:
            cps[1 - cur] = pltpu.async_copy(
                x_hbm.at[0, pl.ds(base + (c + 1) * CHUNK, CHUNK)],
                buf.at[1 - cur],
                sems[1 - cur],
            )
        cps[cur].wait()
        process(c, cur)


@functools.partial(
    pl.kernel,
    out_type=jax.ShapeDtypeStruct((NW, NB1), jnp.int32),
    mesh=_mesh,
    compiler_params=pltpu.CompilerParams(needs_layout_passes=False, use_tc_tiling_on_sc=False),
    scratch_types=[
        pltpu.VMEM((2, CHUNK), jnp.float32),
        pltpu.VMEM((NB1,), jnp.int32),
        pltpu.SemaphoreType.DMA,
        pltpu.SemaphoreType.DMA,
    ],
)
def _sc_coarse_hist(x_hbm, out_hbm, buf, hist, sem0, sem1):
    wid = _worker_id()
    base = wid * PER_W
    _zero_ref(hist, NB1)

    def process(c, cur):
        @plsc.parallel_loop(0, CHUNK, step=L, unroll=8)
        def _(i):
            v = buf[cur, pl.ds(i, L)]
            bits = lax.bitcast_convert_type(v, jnp.int32)
            hi = lax.shift_right_logical(bits, 16)
            plsc.addupdate_scatter(hist, [hi], jnp.ones((L,), jnp.int32))

    _stream_chunks(x_hbm, base, buf, sem0, sem1, process)
    pltpu.sync_copy(hist, out_hbm.at[wid])


@functools.partial(
    pl.kernel,
    out_type=jax.ShapeDtypeStruct((NW, NB2), jnp.int32),
    mesh=_mesh,
    compiler_params=pltpu.CompilerParams(needs_layout_passes=False, use_tc_tiling_on_sc=False),
    scratch_types=[
        pltpu.VMEM((2, CHUNK), jnp.float32),
        pltpu.VMEM((NB2,), jnp.int32),
        pltpu.VMEM((L,), jnp.int32),
        pltpu.SemaphoreType.DMA,
        pltpu.SemaphoreType.DMA,
    ],
)
def _sc_fine_hist(x_hbm, cb_hbm, out_hbm, buf, hist, cbv, sem0, sem1):
    wid = _worker_id()
    base = wid * PER_W
    _zero_ref(hist, NB2)
    pltpu.sync_copy(cb_hbm, cbv)
    cb = cbv[...]
    ones = jnp.ones((L,), jnp.int32)
    lowmask = jnp.full((L,), NB2 - 1, jnp.int32)

    def process(c, cur):
        @plsc.parallel_loop(0, CHUNK, step=L, unroll=8)
        def _(i):
            v = buf[cur, pl.ds(i, L)]
            bits = lax.bitcast_convert_type(v, jnp.int32)
            hi = lax.shift_right_logical(bits, 16)
            lo = jnp.bitwise_and(lax.shift_right_logical(bits, 2), lowmask)
            plsc.addupdate_scatter(hist, [lo], ones, mask=hi == cb)

    _stream_chunks(x_hbm, base, buf, sem0, sem1, process)
    pltpu.sync_copy(hist, out_hbm.at[wid])


@functools.partial(
    pl.kernel,
    out_type=jax.ShapeDtypeStruct((N,), jnp.float32),
    mesh=_mesh,
    compiler_params=pltpu.CompilerParams(needs_layout_passes=False, use_tc_tiling_on_sc=False),
    scratch_types=[
        pltpu.VMEM((2, CHUNK), jnp.float32),
        pltpu.VMEM((2, CHUNK), jnp.float32),
        pltpu.VMEM((L,), jnp.int32),
        pltpu.SemaphoreType.DMA,
        pltpu.SemaphoreType.DMA,
        pltpu.SemaphoreType.DMA,
        pltpu.SemaphoreType.DMA,
    ],
)
def _sc_write_mask(x_hbm, tb_hbm, y_hbm, buf, obuf, tbv, sem0, sem1, osem0, osem1):
    wid = _worker_id()
    base = wid * PER_W
    pltpu.sync_copy(tb_hbm, tbv)
    tb = tbv[...]
    osems = (osem0, osem1)
    ocps = [None, None]

    def process(c, cur):
        if ocps[cur] is not None:
            ocps[cur].wait()

        @plsc.parallel_loop(0, CHUNK, step=L, unroll=8)
        def _(i):
            v = buf[cur, pl.ds(i, L)]
            bits = lax.bitcast_convert_type(v, jnp.int32)
            obuf[cur, pl.ds(i, L)] = (bits >= tb).astype(jnp.float32)
        ocps[cur] = pltpu.async_copy(
            obuf.at[cur], y_hbm.at[pl.ds(base + c * CHUNK, CHUNK)], osems[cur]
        )

    _stream_chunks(x_hbm, base, buf, sem0, sem1, process)
    ocps[0].wait()
    ocps[1].wait()


def _bisect(h, iota, rank, lo0, hi0, steps):
    def body(t, carry):
        lo, hi = carry
        mid = (lo + hi) // 2
        c = jnp.sum(jnp.where(iota > mid, h, 0))
        pred = c < rank
        return jnp.where(pred, lo, mid), jnp.where(pred, mid, hi)

    lo, hi = lax.fori_loop(0, steps, body, (jnp.int32(lo0), jnp.int32(hi0)))
    cnt = jnp.sum(jnp.where(iota > hi, h, 0))
    return hi, cnt


def _tc_coarse_body(h1_ref, out_ref):
    h = jnp.sum(h1_ref[...], axis=0, keepdims=True)
    iota = lax.broadcasted_iota(jnp.int32, (1, NB1), 1)
    cb, _ = _bisect(h, iota, RANK, -1, NB1 - 1, 14)
    out_ref[...] = jnp.full((L,), cb, jnp.int32)


def _tc_fine_body(h1_ref, h2_ref, cb_ref, out_ref):
    cb = jnp.max(cb_ref[...])
    h1 = jnp.sum(h1_ref[...], axis=0, keepdims=True)
    iota1 = lax.broadcasted_iota(jnp.int32, (1, NB1), 1)
    cnt_gt = jnp.sum(jnp.where(iota1 > cb, h1, 0))
    r1 = RANK - cnt_gt
    h2 = jnp.sum(h2_ref[...], axis=0, keepdims=True)
    iota2 = lax.broadcasted_iota(jnp.int32, (1, NB2), 1)
    fb, _ = _bisect(h2, iota2, r1, -1, NB2 - 1, 14)
    tb = lax.shift_left(cb, 16) + lax.shift_left(fb, 2)
    out_ref[...] = jnp.full((L,), tb, jnp.int32)


EX_GRID = 25
EX_BLK = N // EX_GRID


def _tc_extract_kld_body(kref, b_ref, x_ref, kld_ref):
    pi = pl.program_id(0)
    v = b_ref[pl.ds(kref[0], 1), :]
    x_ref[...] = v
    xc = jnp.clip(v, -10.0, 10.0)
    u = jnp.exp(-xc)
    w = 1.0 + u
    t = 0.6931471805599453 - jnp.log(w) - xc * u / w
    s = jnp.sum(t) * (1.0 / N)
    prev = jnp.where(pi == 0, 0.0, kld_ref[0, 0])
    kld_ref[0, 0] = prev + s


def kernel(B, k):
    k_arr = jnp.reshape(k, (1,)).astype(jnp.int32)
    x, kld = pl.pallas_call(
        _tc_extract_kld_body,
        grid_spec=pltpu.PrefetchScalarGridSpec(
            num_scalar_prefetch=1,
            grid=(EX_GRID,),
            in_specs=[pl.BlockSpec((4, EX_BLK), lambda i, kref: (0, i))],
            out_specs=[
                pl.BlockSpec((1, EX_BLK), lambda i, kref: (0, i)),
                pl.BlockSpec(
                    block_shape=(1, 1),
                    index_map=lambda i, kref: (0, 0),
                    memory_space=pltpu.SMEM,
                ),
            ],
        ),
        out_shape=[
            jax.ShapeDtypeStruct((1, N), jnp.float32),
            jax.ShapeDtypeStruct((1, 1), jnp.float32),
        ],
    )(k_arr, B)
    h1 = _sc_coarse_hist(x)
    cb = pl.pallas_call(
        _tc_coarse_body,
        out_shape=jax.ShapeDtypeStruct((L,), jnp.int32),
    )(h1)
    h2 = _sc_fine_hist(x, cb)
    tb = pl.pallas_call(
        _tc_fine_body,
        out_shape=jax.ShapeDtypeStruct((L,), jnp.int32),
    )(h1, h2, cb)
    y = _sc_write_mask(x, tb)
    return y, kld[0, 0]
